# Initial kernel scaffold; baseline (speedup 1.0000x reference)
import jax
import jax.numpy as jnp
from jax.experimental import pallas as pl


def kernel(x, edge_attr, W_mlp_in, b_mlp_in, W_e1, b_e1, W_e2, b_e2, root, conv_bias, gru_W_ih, gru_W_hh, gru_b_ih, gru_b_hh, lstm_W_ih, lstm_W_hh, lstm_b_ih, lstm_b_hh, W_o1, b_o1, W_o2, b_o2, W_o3, b_o3, edge_index, batch):
    raise NotImplementedError("write your pallas kernel here")


# speedup vs baseline: 3.1524x; 3.1524x over previous
import functools

import jax
import jax.numpy as jnp
import numpy as np
from jax import lax
from jax.experimental import pallas as pl
from jax.experimental.pallas import tpu as pltpu
from jax.experimental.pallas import tpu_sc as plsc

N = 10000
E = 160000
D_IN = 128
D_E = 16
H = 32
B = 64

NC = 2
NS = 16
NW = NC * NS
CH = 128
NCHUNK = E // CH
ITERS = (NCHUNK + NW - 1) // NW
NSTRIPE = N // NS

_mesh = plsc.VectorSubcoreMesh(
    core_axis_name="c", subcore_axis_name="s", num_cores=NC, num_subcores=NS)
_sc_params = pltpu.CompilerParams(use_tc_tiling_on_sc=False)


def _h_body(x_ref, w_ref, b_ref, out_ref):
    out_ref[...] = jnp.maximum(
        jnp.dot(x_ref[...], w_ref[...], preferred_element_type=jnp.float32)
        + b_ref[...], 0.0)


def _stage1(x, w, b2d):
    return pl.pallas_call(
        _h_body,
        out_shape=jax.ShapeDtypeStruct((N, H), jnp.float32),
    )(x, w, b2d)


@functools.partial(
    pl.kernel,
    out_type=jax.ShapeDtypeStruct((NCHUNK, CH, H), jnp.float32),
    mesh=_mesh,
    compiler_params=_sc_params,
    scratch_types=[
        pltpu.VMEM((CH,), jnp.int32),
        pltpu.VMEM((CH, H), jnp.float32),
        pltpu.SemaphoreType.DMA,
    ],
)
def _gather_k(h_hbm, src_hbm, out_hbm, idx_v, rows_v, sem):
    w = lax.axis_index("s") * NC + lax.axis_index("c")

    def body(i, carry):
        j = w + i * NW

        @pl.when(j < NCHUNK)
        def _():
            pltpu.sync_copy(src_hbm.at[j], idx_v)
            pltpu.async_copy(h_hbm.at[idx_v], rows_v, sem).wait()
            pltpu.sync_copy(rows_v, out_hbm.at[j])

        return carry

    lax.fori_loop(0, ITERS, body, 0)


EB = 2000


def _edge_body(ea_ref, xj_ref, we1_ref, be1_ref, rm_ref, tm_ref, w2p_ref,
               b2_ref, out_ref):
    ew1 = jnp.maximum(
        jnp.dot(ea_ref[...], we1_ref[...], preferred_element_type=jnp.float32)
        + be1_ref[...], 0.0)
    z = (jnp.dot(ew1, rm_ref[...], preferred_element_type=jnp.float32)
         * jnp.dot(xj_ref[...], tm_ref[...], preferred_element_type=jnp.float32))
    out_ref[...] = (
        jnp.dot(z, w2p_ref[...], preferred_element_type=jnp.float32)
        + jnp.dot(xj_ref[...], b2_ref[...], preferred_element_type=jnp.float32))


def _stage3(ea, xj, we1, be1_2d, w2p, b2):
    k_of = np.arange(H * H) // H
    h_of = np.arange(H * H) % H
    rm = jnp.asarray((np.arange(H)[:, None] == k_of[None, :]).astype(np.float32))
    tm = jnp.asarray((np.arange(H)[:, None] == h_of[None, :]).astype(np.float32))
    grid = E // EB
    return pl.pallas_call(
        _edge_body,
        grid=(grid,),
        in_specs=[
            pl.BlockSpec((EB, D_E), lambda i: (i, 0)),
            pl.BlockSpec((EB, H), lambda i: (i, 0)),
            pl.BlockSpec((D_E, H), lambda i: (0, 0)),
            pl.BlockSpec((1, H), lambda i: (0, 0)),
            pl.BlockSpec((H, H * H), lambda i: (0, 0)),
            pl.BlockSpec((H, H * H), lambda i: (0, 0)),
            pl.BlockSpec((H * H, H), lambda i: (0, 0)),
            pl.BlockSpec((H, H), lambda i: (0, 0)),
        ],
        out_specs=pl.BlockSpec((EB, H), lambda i: (i, 0)),
        out_shape=jax.ShapeDtypeStruct((E, H), jnp.float32),
    )(ea, xj, we1, be1_2d, rm, tm, w2p, b2)


@functools.partial(
    pl.kernel,
    out_type=(
        jax.ShapeDtypeStruct((NC * N, H), jnp.float32),
        jax.ShapeDtypeStruct((NC * N, 16), jnp.float32),
    ),
    mesh=_mesh,
    compiler_params=_sc_params,
    scratch_types=[
        pltpu.VMEM((CH,), jnp.int32),
        pltpu.VMEM((CH, H), jnp.float32),
        pltpu.VMEM((CH, 16), jnp.float32),
        pltpu.VMEM_SHARED((N, H), jnp.float32),
        pltpu.VMEM_SHARED((N, 16), jnp.float32),
    ],
)
def _scatter_k(msg_hbm, dst_hbm, z32_hbm, z16_hbm, ones_hbm,
               agg_hbm, cnt_hbm, idx_v, rows_v, ones_v, agg_sh, cnt_sh):
    c = lax.axis_index("c")
    s = lax.axis_index("s")
    w = s * NC + c

    pltpu.sync_copy(ones_hbm, ones_v)
    pltpu.sync_copy(z32_hbm, agg_sh.at[pl.ds(s * NSTRIPE, NSTRIPE)])
    pltpu.sync_copy(z16_hbm, cnt_sh.at[pl.ds(s * NSTRIPE, NSTRIPE)])
    plsc.subcore_barrier()

    def body(i, carry):
        j = w + i * NW

        @pl.when(j < NCHUNK)
        def _():
            pltpu.sync_copy(dst_hbm.at[j], idx_v)
            pltpu.sync_copy(msg_hbm.at[j], rows_v)
            pltpu.sync_copy(rows_v, agg_sh.at[idx_v], add=True)
            pltpu.sync_copy(ones_v, cnt_sh.at[idx_v], add=True)

        return carry

    lax.fori_loop(0, ITERS, body, 0)
    plsc.subcore_barrier()

    base = c * N + s * NSTRIPE
    pltpu.sync_copy(agg_sh.at[pl.ds(s * NSTRIPE, NSTRIPE)],
                    agg_hbm.at[pl.ds(base, NSTRIPE)])
    pltpu.sync_copy(cnt_sh.at[pl.ds(s * NSTRIPE, NSTRIPE)],
                    cnt_hbm.at[pl.ds(base, NSTRIPE)])


def _final_body(aggp_ref, cntp_ref, h_ref, batch_ref, root_ref, cbias_ref,
                giw_ref, gib_ref, ghw_ref, ghb_ref, lbi_ref, lbh_ref,
                wo1_ref, bo1_ref, wo2_ref, bo2_ref, wo3_ref, bo3_ref, out_ref):
    aggp = aggp_ref[...]
    agg = aggp[:N] + aggp[N:]
    cntp = cntp_ref[...]
    cnt = cntp[:N, 0:1] + cntp[N:, 0:1]
    agg = agg / jnp.maximum(cnt, 1.0)
    h = h_ref[...]
    m_v = jnp.maximum(
        agg + jnp.dot(h, root_ref[...], preferred_element_type=jnp.float32)
        + cbias_ref[...], 0.0)
    gi = jnp.dot(m_v, giw_ref[...], preferred_element_type=jnp.float32) + gib_ref[...]
    gh = jnp.dot(h, ghw_ref[...], preferred_element_type=jnp.float32) + ghb_ref[...]
    r = jax.nn.sigmoid(gi[:, :H] + gh[:, :H])
    zg = jax.nn.sigmoid(gi[:, H:2 * H] + gh[:, H:2 * H])
    n = jnp.tanh(gi[:, 2 * H:] + r * gh[:, 2 * H:])
    h2 = (1.0 - zg) * n + zg * h
    g = lbi_ref[...] + lbh_ref[...]
    ig = jax.nn.sigmoid(g[:, :H])
    gg = jnp.tanh(g[:, 2 * H:3 * H])
    og = jax.nn.sigmoid(g[:, 3 * H:])
    qv = og * jnp.tanh(ig * gg)
    e = jnp.sum(h2 * qv, axis=1, keepdims=True)
    bidx = batch_ref[...]
    oh = (bidx == lax.broadcasted_iota(jnp.int32, (N, B), 1)).astype(jnp.float32)
    masked = jnp.where(oh > 0.0, jnp.broadcast_to(e, (N, B)), -1e30)
    emax = jnp.max(masked, axis=0, keepdims=True)
    a = jnp.exp(e - jnp.sum(oh * emax, axis=1, keepdims=True))
    den = jnp.sum(oh * a, axis=0, keepdims=True)
    a = a / jnp.sum(oh * den, axis=1, keepdims=True)
    r_vec = lax.dot_general(oh, a * h2, (((0,), (0,)), ((), ())),
                            preferred_element_type=jnp.float32)
    q_star = jnp.concatenate(
        [jnp.broadcast_to(qv, (B, H)), r_vec], axis=1)
    y = jnp.maximum(
        jnp.dot(q_star, wo1_ref[...], preferred_element_type=jnp.float32)
        + bo1_ref[...], 0.0)
    y = jnp.maximum(
        jnp.dot(y, wo2_ref[...], preferred_element_type=jnp.float32)
        + bo2_ref[...], 0.0)
    out_ref[...] = (
        jnp.dot(y, wo3_ref[...], preferred_element_type=jnp.float32)
        + bo3_ref[...])


def _stage5(aggp, cntp, h, batch2d, root, cbias, giw, gib, ghw, ghb,
            lbi, lbh, wo1, bo1, wo2, bo2, wo3, bo3):
    return pl.pallas_call(
        _final_body,
        out_shape=jax.ShapeDtypeStruct((B, 1), jnp.float32),
    )(aggp, cntp, h, batch2d, root, cbias, giw, gib, ghw, ghb,
      lbi, lbh, wo1, bo1, wo2, bo2, wo3, bo3)


def kernel(x, edge_attr, W_mlp_in, b_mlp_in, W_e1, b_e1, W_e2, b_e2, root,
           conv_bias, gru_W_ih, gru_W_hh, gru_b_ih, gru_b_hh,
           lstm_W_ih, lstm_W_hh, lstm_b_ih, lstm_b_hh,
           W_o1, b_o1, W_o2, b_o2, W_o3, b_o3, edge_index, batch):
    src = edge_index[0].reshape(NCHUNK, CH)
    dst = edge_index[1].reshape(NCHUNK, CH)

    h = _stage1(x, W_mlp_in, b_mlp_in.reshape(1, H))

    xj = _gather_k(h, src).reshape(E, H)

    msg = _stage3(edge_attr, xj, W_e1, b_e1.reshape(1, H),
                  W_e2.reshape(H * H, H), b_e2.reshape(H, H))

    z32 = jnp.zeros((NSTRIPE, H), jnp.float32)
    z16 = jnp.zeros((NSTRIPE, 16), jnp.float32)
    ones = jnp.ones((CH, 16), jnp.float32)
    aggp, cntp = _scatter_k(msg.reshape(NCHUNK, CH, H), dst, z32, z16, ones)

    y = _stage5(aggp, cntp, h, batch.reshape(N, 1),
                root, conv_bias.reshape(1, H),
                gru_W_ih.T, gru_b_ih.reshape(1, 3 * H),
                gru_W_hh.T, gru_b_hh.reshape(1, 3 * H),
                lstm_b_ih.reshape(1, 4 * H), lstm_b_hh.reshape(1, 4 * H),
                W_o1, b_o1.reshape(1, 512), W_o2, b_o2.reshape(1, 256),
                W_o3, b_o3.reshape(1, 1))
    return y.reshape(-1)


# speedup vs baseline: 3.6452x; 1.1564x over previous
import functools

import jax
import jax.numpy as jnp
import numpy as np
from jax import lax
from jax.experimental import pallas as pl
from jax.experimental.pallas import tpu as pltpu
from jax.experimental.pallas import tpu_sc as plsc

N = 10000
E = 160000
D_IN = 128
D_E = 16
H = 32
B = 64

NC = 2
NS = 16
NW = NC * NS
CH = 128
NCHUNK = E // CH
ITERS = (NCHUNK + NW - 1) // NW
NSTRIPE = N // NS

_mesh = plsc.VectorSubcoreMesh(
    core_axis_name="c", subcore_axis_name="s", num_cores=NC, num_subcores=NS)
_sc_params = pltpu.CompilerParams(use_tc_tiling_on_sc=False)


def _h_body(x_ref, w_ref, b_ref, out_ref):
    out_ref[...] = jnp.maximum(
        jnp.dot(x_ref[...], w_ref[...], preferred_element_type=jnp.float32)
        + b_ref[...], 0.0)


def _stage1(x, w, b2d):
    return pl.pallas_call(
        _h_body,
        out_shape=jax.ShapeDtypeStruct((N, H), jnp.float32),
    )(x, w, b2d)


@functools.partial(
    pl.kernel,
    out_type=jax.ShapeDtypeStruct((NCHUNK, CH, H), jnp.float32),
    mesh=_mesh,
    compiler_params=_sc_params,
    scratch_types=[
        pltpu.VMEM((CH,), jnp.int32),
        pltpu.VMEM((CH, H), jnp.float32),
        pltpu.SemaphoreType.DMA,
    ],
)
def _gather_k(h_hbm, src_hbm, out_hbm, idx_v, rows_v, sem):
    w = lax.axis_index("s") * NC + lax.axis_index("c")

    def body(i, carry):
        j = w + i * NW

        @pl.when(j < NCHUNK)
        def _():
            pltpu.sync_copy(src_hbm.at[j], idx_v)
            pltpu.async_copy(h_hbm.at[idx_v], rows_v, sem).wait()
            pltpu.sync_copy(rows_v, out_hbm.at[j])

        return carry

    lax.fori_loop(0, ITERS, body, 0)


EB = 4000


def _edge_body(ea_ref, xj_ref, we1_ref, be1_ref, rm_ref, w2t_ref,
               b2_ref, out_ref):
    ew1 = jnp.maximum(
        jnp.dot(ea_ref[...], we1_ref[...], preferred_element_type=jnp.float32)
        + be1_ref[...], 0.0)
    xj = xj_ref[...]
    t = jnp.dot(xj, w2t_ref[...], preferred_element_type=jnp.float32)
    p = jnp.dot(ew1, rm_ref[...], preferred_element_type=jnp.float32) * t
    s = p[:, :512] + p[:, 512:]
    s = s[:, :256] + s[:, 256:]
    s = s[:, :128] + s[:, 128:]
    s = s[:, :64] + s[:, 64:]
    out_ref[...] = (
        s[:, :H] + s[:, H:]
        + jnp.dot(xj, b2_ref[...], preferred_element_type=jnp.float32))


def _stage3(ea, xj, we1, be1_2d, w2t, b2):
    k_of = np.arange(H * H) // H
    rm = jnp.asarray((np.arange(H)[:, None] == k_of[None, :]).astype(np.float32))
    grid = E // EB
    return pl.pallas_call(
        _edge_body,
        grid=(grid,),
        in_specs=[
            pl.BlockSpec((EB, D_E), lambda i: (i, 0)),
            pl.BlockSpec((EB, H), lambda i: (i, 0)),
            pl.BlockSpec((D_E, H), lambda i: (0, 0)),
            pl.BlockSpec((1, H), lambda i: (0, 0)),
            pl.BlockSpec((H, H * H), lambda i: (0, 0)),
            pl.BlockSpec((H, H * H), lambda i: (0, 0)),
            pl.BlockSpec((H, H), lambda i: (0, 0)),
        ],
        out_specs=pl.BlockSpec((EB, H), lambda i: (i, 0)),
        out_shape=jax.ShapeDtypeStruct((E, H), jnp.float32),
    )(ea, xj, we1, be1_2d, rm, w2t, b2)


@functools.partial(
    pl.kernel,
    out_type=(
        jax.ShapeDtypeStruct((NC * N, H), jnp.float32),
        jax.ShapeDtypeStruct((NC * N, 16), jnp.float32),
    ),
    mesh=_mesh,
    compiler_params=_sc_params,
    scratch_types=[
        pltpu.VMEM((CH,), jnp.int32),
        pltpu.VMEM((CH, H), jnp.float32),
        pltpu.VMEM((CH, 16), jnp.float32),
        pltpu.VMEM_SHARED((N, H), jnp.float32),
        pltpu.VMEM_SHARED((N, 16), jnp.float32),
    ],
)
def _scatter_k(msg_hbm, dst_hbm, z32_hbm, z16_hbm, ones_hbm,
               agg_hbm, cnt_hbm, idx_v, rows_v, ones_v, agg_sh, cnt_sh):
    c = lax.axis_index("c")
    s = lax.axis_index("s")
    w = s * NC + c

    pltpu.sync_copy(ones_hbm, ones_v)
    pltpu.sync_copy(z32_hbm, agg_sh.at[pl.ds(s * NSTRIPE, NSTRIPE)])
    pltpu.sync_copy(z16_hbm, cnt_sh.at[pl.ds(s * NSTRIPE, NSTRIPE)])
    plsc.subcore_barrier()

    def body(i, carry):
        j = w + i * NW

        @pl.when(j < NCHUNK)
        def _():
            pltpu.sync_copy(dst_hbm.at[j], idx_v)
            pltpu.sync_copy(msg_hbm.at[j], rows_v)
            pltpu.sync_copy(rows_v, agg_sh.at[idx_v], add=True)
            pltpu.sync_copy(ones_v, cnt_sh.at[idx_v], add=True)

        return carry

    lax.fori_loop(0, ITERS, body, 0)
    plsc.subcore_barrier()

    base = c * N + s * NSTRIPE
    pltpu.sync_copy(agg_sh.at[pl.ds(s * NSTRIPE, NSTRIPE)],
                    agg_hbm.at[pl.ds(base, NSTRIPE)])
    pltpu.sync_copy(cnt_sh.at[pl.ds(s * NSTRIPE, NSTRIPE)],
                    cnt_hbm.at[pl.ds(base, NSTRIPE)])


def _final_body(aggp_ref, cntp_ref, h_ref, batch_ref, root_ref, cbias_ref,
                giw_ref, gib_ref, ghw_ref, ghb_ref, lbi_ref, lbh_ref,
                wo1_ref, bo1_ref, wo2_ref, bo2_ref, wo3_ref, bo3_ref, out_ref):
    aggp = aggp_ref[...]
    agg = aggp[:N] + aggp[N:]
    cntp = cntp_ref[...]
    cnt = cntp[:N, 0:1] + cntp[N:, 0:1]
    agg = agg / jnp.maximum(cnt, 1.0)
    h = h_ref[...]
    m_v = jnp.maximum(
        agg + jnp.dot(h, root_ref[...], preferred_element_type=jnp.float32)
        + cbias_ref[...], 0.0)
    gi = jnp.dot(m_v, giw_ref[...], preferred_element_type=jnp.float32) + gib_ref[...]
    gh = jnp.dot(h, ghw_ref[...], preferred_element_type=jnp.float32) + ghb_ref[...]
    r = jax.nn.sigmoid(gi[:, :H] + gh[:, :H])
    zg = jax.nn.sigmoid(gi[:, H:2 * H] + gh[:, H:2 * H])
    n = jnp.tanh(gi[:, 2 * H:] + r * gh[:, 2 * H:])
    h2 = (1.0 - zg) * n + zg * h
    g = lbi_ref[...] + lbh_ref[...]
    ig = jax.nn.sigmoid(g[:, :H])
    gg = jnp.tanh(g[:, 2 * H:3 * H])
    og = jax.nn.sigmoid(g[:, 3 * H:])
    qv = og * jnp.tanh(ig * gg)
    e = jnp.sum(h2 * qv, axis=1, keepdims=True)
    bidx = batch_ref[...]
    oh = (bidx == lax.broadcasted_iota(jnp.int32, (N, B), 1)).astype(jnp.float32)
    masked = jnp.where(oh > 0.0, jnp.broadcast_to(e, (N, B)), -1e30)
    emax = jnp.max(masked, axis=0, keepdims=True)
    a = jnp.exp(e - jnp.sum(oh * emax, axis=1, keepdims=True))
    den = jnp.sum(oh * a, axis=0, keepdims=True)
    a = a / jnp.sum(oh * den, axis=1, keepdims=True)
    r_vec = lax.dot_general(oh, a * h2, (((0,), (0,)), ((), ())),
                            preferred_element_type=jnp.float32)
    q_star = jnp.concatenate(
        [jnp.broadcast_to(qv, (B, H)), r_vec], axis=1)
    y = jnp.maximum(
        jnp.dot(q_star, wo1_ref[...], preferred_element_type=jnp.float32)
        + bo1_ref[...], 0.0)
    y = jnp.maximum(
        jnp.dot(y, wo2_ref[...], preferred_element_type=jnp.float32)
        + bo2_ref[...], 0.0)
    out_ref[...] = (
        jnp.dot(y, wo3_ref[...], preferred_element_type=jnp.float32)
        + bo3_ref[...])


def _stage5(aggp, cntp, h, batch2d, root, cbias, giw, gib, ghw, ghb,
            lbi, lbh, wo1, bo1, wo2, bo2, wo3, bo3):
    return pl.pallas_call(
        _final_body,
        out_shape=jax.ShapeDtypeStruct((B, 1), jnp.float32),
    )(aggp, cntp, h, batch2d, root, cbias, giw, gib, ghw, ghb,
      lbi, lbh, wo1, bo1, wo2, bo2, wo3, bo3)


def kernel(x, edge_attr, W_mlp_in, b_mlp_in, W_e1, b_e1, W_e2, b_e2, root,
           conv_bias, gru_W_ih, gru_W_hh, gru_b_ih, gru_b_hh,
           lstm_W_ih, lstm_W_hh, lstm_b_ih, lstm_b_hh,
           W_o1, b_o1, W_o2, b_o2, W_o3, b_o3, edge_index, batch):
    src = edge_index[0].reshape(NCHUNK, CH)
    dst = edge_index[1].reshape(NCHUNK, CH)

    h = _stage1(x, W_mlp_in, b_mlp_in.reshape(1, H))

    xj = _gather_k(h, src).reshape(E, H)

    w2t = W_e2.reshape(H, H, H).transpose(1, 0, 2).reshape(H, H * H)
    msg = _stage3(edge_attr, xj, W_e1, b_e1.reshape(1, H),
                  w2t, b_e2.reshape(H, H))

    z32 = jnp.zeros((NSTRIPE, H), jnp.float32)
    z16 = jnp.zeros((NSTRIPE, 16), jnp.float32)
    ones = jnp.ones((CH, 16), jnp.float32)
    aggp, cntp = _scatter_k(msg.reshape(NCHUNK, CH, H), dst, z32, z16, ones)

    y = _stage5(aggp, cntp, h, batch.reshape(N, 1),
                root, conv_bias.reshape(1, H),
                gru_W_ih.T, gru_b_ih.reshape(1, 3 * H),
                gru_W_hh.T, gru_b_hh.reshape(1, 3 * H),
                lstm_b_ih.reshape(1, 4 * H), lstm_b_hh.reshape(1, 4 * H),
                W_o1, b_o1.reshape(1, 512), W_o2, b_o2.reshape(1, 256),
                W_o3, b_o3.reshape(1, 1))
    return y.reshape(-1)
